# baseline (speedup 1.0000x reference)
import jax
import jax.numpy as jnp
from jax.experimental import pallas as pl
from jax.experimental.pallas import tpu as pltpu

IN_FEATURES = 1024
N_CLASSES = 100000
C1 = 2000
C2 = 10000
N_TOKENS = 1024
BLK = 1024
LANES = 128
NCH = BLK // LANES
NBLK = (N_CLASSES + BLK - 1) // BLK
B_S1 = C1 // BLK
B_S2 = C2 // BLK
NEG = -1e30


def _update(m_ref, s_ref, cs):
    mx = cs[0]
    for c in cs[1:]:
        mx = jnp.maximum(mx, c)
    mo = m_ref[...]
    mn = jnp.maximum(mo, mx)
    acc = s_ref[...] * jnp.exp(mo - mn)
    for c in cs:
        acc = acc + jnp.exp(c - mn)
    s_ref[...] = acc
    m_ref[...] = mn


def _flash_kernel(x_ref, w_ref, b_ref, tgt_ref, tv_ref, tb_ref,
                  out_ref, loss_ref,
                  lt, m0, s0, m1, s1, m2, s2, pk):
    blk = pl.program_id(0)

    @pl.when(blk == 0)
    def _init():
        for r in (m0, m1, m2):
            r[...] = jnp.full((N_TOKENS, LANES), NEG, jnp.float32)
        for r in (s0, s1, s2, pk):
            r[...] = jnp.zeros((N_TOKENS, LANES), jnp.float32)

    par = jax.lax.rem(blk, 2)

    @pl.when(blk < NBLK)
    def _produce():
        logits = jax.lax.dot_general(
            x_ref[...], w_ref[...].astype(jnp.bfloat16),
            (((1,), (1,)), ((), ())),
            preferred_element_type=jnp.float32)
        lt[par] = logits + b_ref[0]

    j = blk - 1

    def consume(update_fn):
        tile = lt[1 - par]
        cs = [tile[:, i * LANES:(i + 1) * LANES] for i in range(NCH)]
        cls = j * BLK + jax.lax.broadcasted_iota(jnp.int32, (1, BLK), 1)
        clc = [cls[:, i * LANES:(i + 1) * LANES] for i in range(NCH)]
        tgt = tgt_ref[...]
        pk[...] = pk[...] + sum(
            jnp.where(c == tgt, v, 0.0) for c, v in zip(clc, cs))
        update_fn(clc, cs)

    @pl.when((blk >= 1) & (j < B_S1))
    def _pure0():
        consume(lambda clc, cs: _update(m0, s0, cs))

    @pl.when(j == B_S1)
    def _straddle01():
        def f(clc, cs):
            _update(m0, s0,
                    [jnp.where(c < C1, v, NEG) for c, v in zip(clc, cs)])
            _update(m1, s1,
                    [jnp.where(c >= C1, v, NEG) for c, v in zip(clc, cs)])
        consume(f)

    @pl.when((j > B_S1) & (j < B_S2))
    def _pure1():
        consume(lambda clc, cs: _update(m1, s1, cs))

    @pl.when(j == B_S2)
    def _straddle12():
        def f(clc, cs):
            _update(m1, s1,
                    [jnp.where(c < C2, v, NEG) for c, v in zip(clc, cs)])
            _update(m2, s2,
                    [jnp.where(c >= C2, v, NEG) for c, v in zip(clc, cs)])
        consume(f)

    @pl.when((j > B_S2) & (j < NBLK - 1))
    def _pure2():
        consume(lambda clc, cs: _update(m2, s2, cs))

    @pl.when(j == NBLK - 1)
    def _edge():
        consume(lambda clc, cs: _update(
            m2, s2, [jnp.where(c < N_CLASSES, v, NEG)
                     for c, v in zip(clc, cs)]))

    @pl.when(blk == NBLK)
    def _fini():
        def lse_of(m_ref, s_ref):
            mp = m_ref[...]
            mt = jnp.max(mp, axis=1, keepdims=True)
            st = jnp.sum(s_ref[...] * jnp.exp(mp - mt), axis=1, keepdims=True)
            return mt, st

        mt0, st0 = lse_of(m0, s0)
        mt1, st1 = lse_of(m1, s1)
        mt2, st2 = lse_of(m2, s2)

        tlog = jax.lax.dot_general(
            x_ref[...], tv_ref[...].astype(jnp.bfloat16),
            (((1,), (1,)), ((), ())),
            preferred_element_type=jnp.float32) + tb_ref[...]
        tmax = jnp.max(tlog, axis=1, keepdims=True)
        mh = jnp.maximum(mt0, tmax)
        sh = st0 * jnp.exp(mt0 - mh) + jnp.sum(jnp.exp(tlog - mh),
                                               axis=1, keepdims=True)
        lse0 = mh + jnp.log(sh)
        lse1 = mt1 + jnp.log(st1)
        lse2 = mt2 + jnp.log(st2)

        p = jnp.sum(pk[...], axis=1, keepdims=True)
        t = tgt_ref[...]
        is0 = t < C1
        is1 = (t >= C1) & (t < C2)
        head_pick = jnp.where(is0, p, jnp.where(is1, tlog[:, 0:1],
                                                tlog[:, 1:2]))
        tail_part = jnp.where(is0, 0.0, p - jnp.where(is1, lse1, lse2))
        out = head_pick - lse0 + tail_part
        out_ref[...] = out
        loss_ref[...] = jnp.zeros((1, 1), jnp.float32) - jnp.mean(out)


def kernel(x, target, weight, bias, tail_vectors, tail_bias):
    xb = x.astype(jnp.bfloat16)
    bias_p = jnp.pad(bias, (0, NBLK * BLK - N_CLASSES)).reshape(NBLK, 1, BLK)
    tgt2 = target.astype(jnp.int32).reshape(N_TOKENS, 1)
    tb2 = tail_bias.reshape(1, 2)
    last = NBLK - 1
    out, loss = pl.pallas_call(
        _flash_kernel,
        grid=(NBLK + 1,),
        in_specs=[
            pl.BlockSpec((N_TOKENS, IN_FEATURES), lambda b: (0, 0)),
            pl.BlockSpec((BLK, IN_FEATURES), lambda b: (jnp.minimum(b, last), 0)),
            pl.BlockSpec((1, 1, BLK), lambda b: (jnp.minimum(b, last), 0, 0)),
            pl.BlockSpec((N_TOKENS, 1), lambda b: (0, 0)),
            pl.BlockSpec((2, IN_FEATURES), lambda b: (0, 0)),
            pl.BlockSpec((1, 2), lambda b: (0, 0)),
        ],
        out_specs=[
            pl.BlockSpec((N_TOKENS, 1), lambda b: (0, 0)),
            pl.BlockSpec((1, 1), lambda b: (0, 0)),
        ],
        out_shape=[
            jax.ShapeDtypeStruct((N_TOKENS, 1), jnp.float32),
            jax.ShapeDtypeStruct((1, 1), jnp.float32),
        ],
        scratch_shapes=[
            pltpu.VMEM((2, N_TOKENS, BLK), jnp.float32),
            pltpu.VMEM((N_TOKENS, LANES), jnp.float32),
            pltpu.VMEM((N_TOKENS, LANES), jnp.float32),
            pltpu.VMEM((N_TOKENS, LANES), jnp.float32),
            pltpu.VMEM((N_TOKENS, LANES), jnp.float32),
            pltpu.VMEM((N_TOKENS, LANES), jnp.float32),
            pltpu.VMEM((N_TOKENS, LANES), jnp.float32),
            pltpu.VMEM((N_TOKENS, LANES), jnp.float32),
        ],
        compiler_params=pltpu.CompilerParams(
            dimension_semantics=("arbitrary",)),
    )(xb, weight, bias_p, tgt2, tail_vectors, tb2)
    return out.reshape(N_TOKENS), loss[0, 0]


# speedup vs baseline: 1.0153x; 1.0153x over previous
import jax
import jax.numpy as jnp
from jax.experimental import pallas as pl
from jax.experimental.pallas import tpu as pltpu

IN_FEATURES = 1024
N_CLASSES = 100000
C1 = 2000
C2 = 10000
N_TOKENS = 1024
BLK = 1024
LANES = 128
NCH = BLK // LANES
NBLK = (N_CLASSES + BLK - 1) // BLK
B_S1 = C1 // BLK
B_S2 = C2 // BLK
NEG = -1e30


def _flash_kernel(x_ref, w_ref, b_ref, tgt_ref, tv_ref, tb_ref,
                  out_ref, loss_ref, lt, ms, ss, pk):
    blk = pl.program_id(0)
    par = jax.lax.rem(blk, 2)
    j = blk - 1

    def produce():
        logits = jax.lax.dot_general(
            x_ref[...], w_ref[...].astype(jnp.bfloat16),
            (((1,), (1,)), ((), ())),
            preferred_element_type=jnp.float32)
        lt[par] = logits + b_ref[0]

    def consume_chunks():
        tile = lt[1 - par]
        cs = [tile[:, i * LANES:(i + 1) * LANES] for i in range(NCH)]
        cls = j * BLK + jax.lax.broadcasted_iota(jnp.int32, (1, BLK), 1)
        clc = [cls[:, i * LANES:(i + 1) * LANES] for i in range(NCH)]
        tgt = tgt_ref[...]
        pk[...] = pk[...] + sum(
            jnp.where(c == tgt, v, 0.0) for c, v in zip(clc, cs))
        return clc, cs

    def upd(r, cs):
        mx = cs[0]
        for c in cs[1:]:
            mx = jnp.maximum(mx, c)
        mo = ms[r]
        mn = jnp.maximum(mo, mx)
        acc = ss[r] * jnp.exp(mo - mn)
        for c in cs:
            acc = acc + jnp.exp(c - mn)
        ss[r] = acc
        ms[r] = mn

    @pl.when(blk == 0)
    def _first():
        ms[...] = jnp.full((3, N_TOKENS, LANES), NEG, jnp.float32)
        ss[...] = jnp.zeros((3, N_TOKENS, LANES), jnp.float32)
        pk[...] = jnp.zeros((N_TOKENS, LANES), jnp.float32)
        produce()

    @pl.when((blk >= 1) & (blk < NBLK) & (j != B_S1) & (j != B_S2))
    def _steady():
        produce()
        clc, cs = consume_chunks()
        r = (j > B_S1).astype(jnp.int32) + (j > B_S2).astype(jnp.int32)
        upd(r, cs)

    @pl.when(j == B_S1)
    def _straddle01():
        produce()
        clc, cs = consume_chunks()
        upd(0, [jnp.where(c < C1, v, NEG) for c, v in zip(clc, cs)])
        upd(1, [jnp.where(c >= C1, v, NEG) for c, v in zip(clc, cs)])

    @pl.when(j == B_S2)
    def _straddle12():
        produce()
        clc, cs = consume_chunks()
        upd(1, [jnp.where(c < C2, v, NEG) for c, v in zip(clc, cs)])
        upd(2, [jnp.where(c >= C2, v, NEG) for c, v in zip(clc, cs)])

    @pl.when(blk == NBLK)
    def _drain_fini():
        clc, cs = consume_chunks()
        upd(2, [jnp.where(c < N_CLASSES, v, NEG) for c, v in zip(clc, cs)])

        def lse_of(r):
            mp = ms[r]
            mt = jnp.max(mp, axis=1, keepdims=True)
            st = jnp.sum(ss[r] * jnp.exp(mp - mt), axis=1, keepdims=True)
            return mt, st

        mt0, st0 = lse_of(0)
        mt1, st1 = lse_of(1)
        mt2, st2 = lse_of(2)

        tlog = jax.lax.dot_general(
            x_ref[...], tv_ref[...].astype(jnp.bfloat16),
            (((1,), (1,)), ((), ())),
            preferred_element_type=jnp.float32) + tb_ref[...]
        tmax = jnp.max(tlog, axis=1, keepdims=True)
        mh = jnp.maximum(mt0, tmax)
        sh = st0 * jnp.exp(mt0 - mh) + jnp.sum(jnp.exp(tlog - mh),
                                               axis=1, keepdims=True)
        lse0 = mh + jnp.log(sh)
        lse1 = mt1 + jnp.log(st1)
        lse2 = mt2 + jnp.log(st2)

        p = jnp.sum(pk[...], axis=1, keepdims=True)
        t = tgt_ref[...]
        is0 = t < C1
        is1 = (t >= C1) & (t < C2)
        head_pick = jnp.where(is0, p, jnp.where(is1, tlog[:, 0:1],
                                                tlog[:, 1:2]))
        tail_part = jnp.where(is0, 0.0, p - jnp.where(is1, lse1, lse2))
        out = head_pick - lse0 + tail_part
        out_ref[...] = out
        loss_ref[...] = jnp.zeros((1, 1), jnp.float32) - jnp.mean(out)


def kernel(x, target, weight, bias, tail_vectors, tail_bias):
    xb = x.astype(jnp.bfloat16)
    bias_p = jnp.pad(bias, (0, NBLK * BLK - N_CLASSES)).reshape(NBLK, 1, BLK)
    tgt2 = target.astype(jnp.int32).reshape(N_TOKENS, 1)
    tb2 = tail_bias.reshape(1, 2)
    last = NBLK - 1
    out, loss = pl.pallas_call(
        _flash_kernel,
        grid=(NBLK + 1,),
        in_specs=[
            pl.BlockSpec((N_TOKENS, IN_FEATURES), lambda b: (0, 0)),
            pl.BlockSpec((BLK, IN_FEATURES), lambda b: (jnp.minimum(b, last), 0)),
            pl.BlockSpec((1, 1, BLK), lambda b: (jnp.minimum(b, last), 0, 0)),
            pl.BlockSpec((N_TOKENS, 1), lambda b: (0, 0)),
            pl.BlockSpec((2, IN_FEATURES), lambda b: (0, 0)),
            pl.BlockSpec((1, 2), lambda b: (0, 0)),
        ],
        out_specs=[
            pl.BlockSpec((N_TOKENS, 1), lambda b: (0, 0)),
            pl.BlockSpec((1, 1), lambda b: (0, 0)),
        ],
        out_shape=[
            jax.ShapeDtypeStruct((N_TOKENS, 1), jnp.float32),
            jax.ShapeDtypeStruct((1, 1), jnp.float32),
        ],
        scratch_shapes=[
            pltpu.VMEM((2, N_TOKENS, BLK), jnp.float32),
            pltpu.VMEM((3, N_TOKENS, LANES), jnp.float32),
            pltpu.VMEM((3, N_TOKENS, LANES), jnp.float32),
            pltpu.VMEM((N_TOKENS, LANES), jnp.float32),
        ],
        compiler_params=pltpu.CompilerParams(
            dimension_semantics=("arbitrary",)),
    )(xb, weight, bias_p, tgt2, tail_vectors, tb2)
    return out.reshape(N_TOKENS), loss[0, 0]


# speedup vs baseline: 1.1292x; 1.1121x over previous
import jax
import jax.numpy as jnp
from jax.experimental import pallas as pl
from jax.experimental.pallas import tpu as pltpu

IN_FEATURES = 1024
N_CLASSES = 100000
C1 = 2000
C2 = 10000
N_TOKENS = 1024
BLK = 1024
LANES = 128
SUB = 256
NSUB = BLK // SUB
NCH = BLK // LANES
NBLK = (N_CLASSES + BLK - 1) // BLK
B_S1 = C1 // BLK
B_S2 = C2 // BLK
NEG = -1e30


def _update(m_ref, s_ref, cs):
    mx = cs[0]
    for c in cs[1:]:
        mx = jnp.maximum(mx, c)
    mo = m_ref[...]
    mn = jnp.maximum(mo, mx)
    acc = s_ref[...] * jnp.exp(mo - mn)
    for c in cs:
        acc = acc + jnp.exp(c - mn)
    s_ref[...] = acc
    m_ref[...] = mn


def _flash_kernel(x_ref, w_ref, b_ref, tgt_ref, tv_ref, tb_ref,
                  out_ref, loss_ref,
                  m0, s0, m1, s1, m2, s2, pk):
    blk = pl.program_id(0)

    @pl.when(blk == 0)
    def _init():
        for r in (m0, m1, m2):
            r[...] = jnp.full((N_TOKENS, LANES), NEG, jnp.float32)
        for r in (s0, s1, s2, pk):
            r[...] = jnp.zeros((N_TOKENS, LANES), jnp.float32)

    cs = []
    for i in range(NSUB):
        sub = jax.lax.dot_general(
            x_ref[...], w_ref[i * SUB:(i + 1) * SUB, :].astype(jnp.bfloat16),
            (((1,), (1,)), ((), ())),
            preferred_element_type=jnp.float32)
        sub = sub + b_ref[0][:, i * SUB:(i + 1) * SUB]
        cs.append(sub[:, 0:LANES])
        cs.append(sub[:, LANES:2 * LANES])

    cls = blk * BLK + jax.lax.broadcasted_iota(jnp.int32, (1, BLK), 1)
    clc = [cls[:, i * LANES:(i + 1) * LANES] for i in range(NCH)]

    tgt = tgt_ref[...]
    pk[...] = pk[...] + sum(
        jnp.where(c == tgt, v, 0.0) for c, v in zip(clc, cs))

    @pl.when(blk < B_S1)
    def _pure0():
        _update(m0, s0, cs)

    @pl.when(blk == B_S1)
    def _straddle01():
        _update(m0, s0, [jnp.where(c < C1, v, NEG) for c, v in zip(clc, cs)])
        _update(m1, s1, [jnp.where(c >= C1, v, NEG) for c, v in zip(clc, cs)])

    @pl.when((blk > B_S1) & (blk < B_S2))
    def _pure1():
        _update(m1, s1, cs)

    @pl.when(blk == B_S2)
    def _straddle12():
        _update(m1, s1, [jnp.where(c < C2, v, NEG) for c, v in zip(clc, cs)])
        _update(m2, s2, [jnp.where(c >= C2, v, NEG) for c, v in zip(clc, cs)])

    @pl.when((blk > B_S2) & (blk < NBLK - 1))
    def _pure2():
        _update(m2, s2, cs)

    @pl.when(blk == NBLK - 1)
    def _edge():
        _update(m2, s2,
                [jnp.where(c < N_CLASSES, v, NEG) for c, v in zip(clc, cs)])

    @pl.when(blk == NBLK - 1)
    def _fini():
        def lse_of(m_ref, s_ref):
            mp = m_ref[...]
            mt = jnp.max(mp, axis=1, keepdims=True)
            st = jnp.sum(s_ref[...] * jnp.exp(mp - mt), axis=1, keepdims=True)
            return mt, st

        mt0, st0 = lse_of(m0, s0)
        mt1, st1 = lse_of(m1, s1)
        mt2, st2 = lse_of(m2, s2)

        tlog = jax.lax.dot_general(
            x_ref[...], tv_ref[...].astype(jnp.bfloat16),
            (((1,), (1,)), ((), ())),
            preferred_element_type=jnp.float32) + tb_ref[...]
        tmax = jnp.max(tlog, axis=1, keepdims=True)
        mh = jnp.maximum(mt0, tmax)
        sh = st0 * jnp.exp(mt0 - mh) + jnp.sum(jnp.exp(tlog - mh),
                                               axis=1, keepdims=True)
        lse0 = mh + jnp.log(sh)
        lse1 = mt1 + jnp.log(st1)
        lse2 = mt2 + jnp.log(st2)

        p = jnp.sum(pk[...], axis=1, keepdims=True)
        t = tgt_ref[...]
        is0 = t < C1
        is1 = (t >= C1) & (t < C2)
        head_pick = jnp.where(is0, p, jnp.where(is1, tlog[:, 0:1],
                                                tlog[:, 1:2]))
        tail_part = jnp.where(is0, 0.0, p - jnp.where(is1, lse1, lse2))
        out = head_pick - lse0 + tail_part
        out_ref[...] = out
        loss_ref[...] = jnp.zeros((1, 1), jnp.float32) - jnp.mean(out)


def kernel(x, target, weight, bias, tail_vectors, tail_bias):
    xb = x.astype(jnp.bfloat16)
    bias_p = jnp.pad(bias, (0, NBLK * BLK - N_CLASSES)).reshape(NBLK, 1, BLK)
    tgt2 = target.astype(jnp.int32).reshape(N_TOKENS, 1)
    tb2 = tail_bias.reshape(1, 2)
    out, loss = pl.pallas_call(
        _flash_kernel,
        grid=(NBLK,),
        in_specs=[
            pl.BlockSpec((N_TOKENS, IN_FEATURES), lambda b: (0, 0)),
            pl.BlockSpec((BLK, IN_FEATURES), lambda b: (b, 0)),
            pl.BlockSpec((1, 1, BLK), lambda b: (b, 0, 0)),
            pl.BlockSpec((N_TOKENS, 1), lambda b: (0, 0)),
            pl.BlockSpec((2, IN_FEATURES), lambda b: (0, 0)),
            pl.BlockSpec((1, 2), lambda b: (0, 0)),
        ],
        out_specs=[
            pl.BlockSpec((N_TOKENS, 1), lambda b: (0, 0)),
            pl.BlockSpec((1, 1), lambda b: (0, 0)),
        ],
        out_shape=[
            jax.ShapeDtypeStruct((N_TOKENS, 1), jnp.float32),
            jax.ShapeDtypeStruct((1, 1), jnp.float32),
        ],
        scratch_shapes=[
            pltpu.VMEM((N_TOKENS, LANES), jnp.float32),
            pltpu.VMEM((N_TOKENS, LANES), jnp.float32),
            pltpu.VMEM((N_TOKENS, LANES), jnp.float32),
            pltpu.VMEM((N_TOKENS, LANES), jnp.float32),
            pltpu.VMEM((N_TOKENS, LANES), jnp.float32),
            pltpu.VMEM((N_TOKENS, LANES), jnp.float32),
            pltpu.VMEM((N_TOKENS, LANES), jnp.float32),
        ],
        compiler_params=pltpu.CompilerParams(
            dimension_semantics=("arbitrary",)),
    )(xb, weight, bias_p, tgt2, tail_vectors, tb2)
    return out.reshape(N_TOKENS), loss[0, 0]
